# Initial kernel scaffold; baseline (speedup 1.0000x reference)
import jax
import jax.numpy as jnp
from jax.experimental import pallas as pl


def kernel(large_pred, medium_pred, small_pred, boxes, labels, scale_mask):
    raise NotImplementedError("write your pallas kernel here")


# speedup vs baseline: 6.1764x; 6.1764x over previous
import jax
import jax.numpy as jnp
from jax import lax
from jax.experimental import pallas as pl
from jax.experimental.pallas import tpu as pltpu
from jax.experimental.pallas import tpu_sc as plsc

L = 16
NC, NS = 2, 16
B, NB, C = 32, 20, 80
GRIDS = (13, 26, 52)
ANCH = (
    ((3.625, 2.8125), (4.875, 6.1875), (11.65625, 10.1875)),
    ((1.875, 3.8125), (3.875, 2.8125), (3.6875, 7.4375)),
    ((1.25, 1.625), (2.0, 3.75), (4.125, 2.875)),
)
SCALE = []
for _G in GRIDS:
    _N = B * 3 * _G * _G
    _ch = _N // 32
    SCALE.append((_G, _N, _ch, (_ch + L - 1) // L, (_ch + 127) // 128))

LN2 = 0.6931471805599453
SQRTH = 1.41421356237
_LOGP = (7.0376836292e-2, -1.1514610310e-1, 1.1676998740e-1, -1.2420140846e-1,
         1.4249322787e-1, -1.6668057665e-1, 2.0000714765e-1, -2.4999993993e-1,
         3.3333331174e-1)


def _io():
    return lax.iota(jnp.int32, L)


def _ln(v):
    bits = lax.bitcast_convert_type(v, jnp.int32)
    e = lax.shift_right_arithmetic(bits, 23) - 127
    m = lax.bitcast_convert_type(
        (bits & jnp.int32(0x007FFFFF)) | jnp.int32(0x3F800000), jnp.float32)
    big = m > SQRTH
    m = jnp.where(big, m * 0.5, m)
    e = e + jnp.where(big, 1, 0)
    t = m - 1.0
    p = jnp.full((L,), _LOGP[0], jnp.float32)
    for c in _LOGP[1:]:
        p = p * t + c
    y = t + t * t * (t * p - 0.5)
    return e.astype(jnp.float32) * LN2 + y


def _softplus(x):
    return jnp.maximum(x, 0.0) + _ln(1.0 + jnp.exp(-jnp.abs(x)))


def _body(lp, mp, sp, bxh, lbh, out,
          idxd, dg, idxs, gaths, bx, lbv, cellv, lkeyv, basev, tvf,
          winv, lwinv, outv, semd, sems):
    cid = lax.axis_index("c")
    sid = lax.axis_index("s")
    wid = sid * NC + cid
    b = wid
    io = _io()
    z16 = jnp.zeros((L,), jnp.int32)
    preds = (lp, mp, sp)

    pltpu.sync_copy(bxh.at[pl.ds(b * 80, 80)], bx)
    pltpu.sync_copy(lbh.at[pl.ds(b * 32, 32)], lbv)

    def dense_phase(pref, Ns, chunk, nv, K):
        row0 = wid * chunk

        def fill(i, _):
            row = jnp.minimum(row0 + i * L + io, Ns - 1)
            idxd[pl.ds(i * L, L)] = row * 85 + 4
            return 0
        lax.fori_loop(0, nv, fill, 0)
        for j in range(nv * L, K * 128, L):
            idxd[pl.ds(j, L)] = z16 + 4
        cps = []
        for j in range(K):
            cp = pltpu.make_async_copy(
                pref.at[idxd.at[pl.ds(j * 128, 128)]],
                dg.at[pl.ds(j * 128, 128)], semd)
            cp.start()
            cps.append(cp)
        for cp in cps:
            cp.wait()

        def red(i, acc):
            v = dg[pl.ds(i * L, L)]
            msk = (i * L + io) < chunk
            return acc + jnp.where(msk, _softplus(v), 0.0)
        return lax.fori_loop(0, nv, red, jnp.zeros((L,), jnp.float32))

    def sparse_phase(pref, G, anc, accs):
        GG3 = 3 * G * G
        fG = jnp.float32(G)
        for c in range(2):
            j = jnp.minimum(c * L + io, NB - 1)
            valid = (c * L + io) < NB
            x = plsc.load_gather(bx, [j * 4])
            y = plsc.load_gather(bx, [j * 4 + 1])
            w = plsc.load_gather(bx, [j * 4 + 2])
            h = plsc.load_gather(bx, [j * 4 + 3])
            lab = lbv[pl.ds(c * L, L)]
            gx = jnp.minimum((x * fG).astype(jnp.int32), G - 1)
            gy = jnp.minimum((y * fG).astype(jnp.int32), G - 1)
            bw = w * fG
            bh = h * fG
            best = jnp.zeros((L,), jnp.float32) - 1.0
            a = z16
            for k in range(3):
                aw, ah = anc[k]
                inter = jnp.minimum(bw, aw) * jnp.minimum(bh, ah)
                iou = inter / (bw * bh + aw * ah - inter + 1e-16)
                upd = iou > best
                a = jnp.where(upd, k, a)
                best = jnp.where(upd, iou, best)
            aws = jnp.where(a == 0, anc[0][0], jnp.where(a == 1, anc[1][0], anc[2][0]))
            ahs = jnp.where(a == 0, anc[0][1], jnp.where(a == 1, anc[1][1], anc[2][1]))
            xc = x * fG - gx.astype(jnp.float32)
            yc = y * fG - gy.astype(jnp.float32)
            wc = _ln(bw / aws + 1e-16)
            hc = _ln(bh / ahs + 1e-16)
            cell = (a * G + gy) * G + gx
            cell_eff = jnp.where(valid, cell, -1 - (c * L + io))
            cellv[pl.ds(c * L, L)] = cell_eff
            lkeyv[pl.ds(c * L, L)] = cell_eff * 128 + lab
            basev[pl.ds(c * L, L)] = (b * GG3 + cell) * 85
            tvf[pl.ds(c * L, L)] = xc
            tvf[pl.ds(48 + c * L, L)] = yc
            tvf[pl.ds(96 + c * L, L)] = wc
            tvf[pl.ds(144 + c * L, L)] = hc

        def dedup(n, _):
            c0 = cellv[pl.ds(0, L)]
            c1 = cellv[pl.ds(L, L)]
            k0 = lkeyv[pl.ds(0, L)]
            k1 = lkeyv[pl.ds(L, L)]
            cn = plsc.load_gather(cellv, [z16 + n])
            kn = plsc.load_gather(lkeyv, [z16 + n])
            lat0 = io > n
            lat1 = (io + L) > n
            coll = jnp.sum(jnp.where((c0 == cn) & lat0, 1, 0)
                           + jnp.where((c1 == cn) & lat1, 1, 0))
            lcoll = jnp.sum(jnp.where((k0 == kn) & lat0, 1, 0)
                            + jnp.where((k1 == kn) & lat1, 1, 0))
            m0 = io == 0
            plsc.store_scatter(winv, [z16 + n],
                               jnp.where(coll == 0, 1.0, 0.0) + jnp.zeros((L,), jnp.float32),
                               mask=m0)
            plsc.store_scatter(lwinv, [z16 + n],
                               jnp.where(lcoll == 0, 1.0, 0.0) + jnp.zeros((L,), jnp.float32),
                               mask=m0)
            return 0
        lax.fori_loop(0, NB, dedup, 0)

        def fill2(n, _):
            bn = plsc.load_gather(basev, [z16 + n])
            lb = plsc.load_gather(lbv, [z16 + n])
            for k in range(5):
                idxs[pl.ds(n * 96 + k * L, L)] = bn + io + k * L
            off5 = jnp.where(io < 5, io + 80, jnp.where(io == 5, 5 + lb, 84))
            idxs[pl.ds(n * 96 + 80, L)] = bn + off5
            return 0
        lax.fori_loop(0, NB, fill2, 0)

        cps = []
        for n in range(NB):
            cp = pltpu.make_async_copy(
                pref.at[idxs.at[pl.ds(n * 96, 96)]],
                gaths.at[pl.ds(n * 96, 96)], sems)
            cp.start()
            cps.append(cp)
        for cp in cps:
            cp.wait()

        def corr(n, accs):
            loc, obj, cls = accs
            g = [gaths[pl.ds(n * 96 + k * L, L)] for k in range(6)]
            s = [_softplus(gk) for gk in g]
            tv = plsc.load_gather(tvf, [jnp.minimum(io, 3) * 48 + n])
            wn = plsc.load_gather(winv, [z16 + n])
            lwn = plsc.load_gather(lwinv, [z16 + n])
            d = g[0] - tv
            loc = loc + wn * jnp.where(io < 4, d * d, 0.0)
            obj = obj + wn * jnp.where(io == 4, 0.5 * s[0] - g[0], 0.0)
            cb = (jnp.where(io >= 5, s[0], 0.0) + s[1] + s[2] + s[3] + s[4]
                  + jnp.where(io < 5, s[5], 0.0))
            cls = cls + wn * cb - lwn * jnp.where(io == 5, g[5], 0.0)
            return (loc, obj, cls)
        return lax.fori_loop(0, NB, corr, accs)

    zf = jnp.zeros((L,), jnp.float32)
    dense = zf
    accs = (zf, zf, zf)
    for si in range(3):
        G, Ns, ch, nv, K = SCALE[si]
        dense = dense + dense_phase(preds[si], Ns, ch, nv, K)
        accs = sparse_phase(preds[si], G, ANCH[si], accs)
    loc, obj, cls = accs
    obj = obj + 0.5 * dense
    ls = jnp.sum(loc)
    os_ = jnp.sum(obj)
    cs = jnp.sum(cls)
    outv[...] = (jnp.where(io == 0, ls, 0.0) + jnp.where(io == 1, os_, 0.0)
                 + jnp.where(io == 2, cs, 0.0))
    pltpu.sync_copy(outv, out.at[wid])


_MESH = plsc.VectorSubcoreMesh(core_axis_name="c", subcore_axis_name="s",
                               num_cores=NC, num_subcores=NS)

_kfn = pl.kernel(
    _body,
    out_type=jax.ShapeDtypeStruct((B, L), jnp.float32),
    mesh=_MESH,
    compiler_params=pltpu.CompilerParams(needs_layout_passes=False),
    scratch_types=[
        pltpu.VMEM((8192,), jnp.int32),
        pltpu.VMEM((8192,), jnp.float32),
        pltpu.VMEM((1920,), jnp.int32),
        pltpu.VMEM((1920,), jnp.float32),
        pltpu.VMEM((80,), jnp.float32),
        pltpu.VMEM((32,), jnp.int32),
        pltpu.VMEM((32,), jnp.int32),
        pltpu.VMEM((32,), jnp.int32),
        pltpu.VMEM((32,), jnp.int32),
        pltpu.VMEM((192,), jnp.float32),
        pltpu.VMEM((32,), jnp.float32),
        pltpu.VMEM((32,), jnp.float32),
        pltpu.VMEM((L,), jnp.float32),
        pltpu.SemaphoreType.DMA,
        pltpu.SemaphoreType.DMA,
    ],
)


def kernel(large_pred, medium_pred, small_pred, boxes, labels, scale_mask):
    lpf = large_pred.reshape(-1)
    mpf = medium_pred.reshape(-1)
    spf = small_pred.reshape(-1)
    bxf = boxes.reshape(-1)
    lbp = jnp.zeros((B, 32), jnp.int32).at[:, :NB].set(
        labels.astype(jnp.int32)).reshape(-1)
    parts = _kfn(lpf, mpf, spf, bxf, lbp)
    loc = jnp.sum(parts[:, 0])
    obj = jnp.sum(parts[:, 1])
    cls = jnp.sum(parts[:, 2])
    total = 5.0 * loc + obj + cls
    return jnp.stack([total, loc, obj, cls])


# speedup vs baseline: 14.1818x; 2.2961x over previous
import jax
import jax.numpy as jnp
from jax import lax
from jax.experimental import pallas as pl
from jax.experimental.pallas import tpu as pltpu
from jax.experimental.pallas import tpu_sc as plsc

L = 16
NC, NS = 2, 16
B, NB, C = 32, 20, 80
GRIDS = (13, 26, 52)
ANCH = (
    ((3.625, 2.8125), (4.875, 6.1875), (11.65625, 10.1875)),
    ((1.875, 3.8125), (3.875, 2.8125), (3.6875, 7.4375)),
    ((1.25, 1.625), (2.0, 3.75), (4.125, 2.875)),
)
SCALE = []
for _G in GRIDS:
    _N = 3 * _G * _G * B
    _ch = _N // 32
    SCALE.append((_G, _N, _ch, (_ch + L - 1) // L, (_ch + 127) // 128))

LN2 = 0.6931471805599453
SQRTH = 1.41421356237
_LOGP = (7.0376836292e-2, -1.1514610310e-1, 1.1676998740e-1, -1.2420140846e-1,
         1.4249322787e-1, -1.6668057665e-1, 2.0000714765e-1, -2.4999993993e-1,
         3.3333331174e-1)


def _io():
    return lax.iota(jnp.int32, L)


def _ln(v):
    bits = lax.bitcast_convert_type(v, jnp.int32)
    e = lax.shift_right_arithmetic(bits, 23) - 127
    m = lax.bitcast_convert_type(
        (bits & jnp.int32(0x007FFFFF)) | jnp.int32(0x3F800000), jnp.float32)
    big = m > SQRTH
    m = jnp.where(big, m * 0.5, m)
    e = e + jnp.where(big, 1, 0)
    t = m - 1.0
    p = jnp.full((L,), _LOGP[0], jnp.float32)
    for c in _LOGP[1:]:
        p = p * t + c
    y = t + t * t * (t * p - 0.5)
    return e.astype(jnp.float32) * LN2 + y


def _softplus(x):
    return jnp.maximum(x, 0.0) + _ln(1.0 + jnp.exp(-jnp.abs(x)))


def _body(lp, mp, sp, bxh, lbh, out,
          idxd, dg, idxs, gaths, bx, lbv, cellv, lkeyv, basev, tvf,
          winv, lwinv, outv, semd, sems):
    cid = lax.axis_index("c")
    sid = lax.axis_index("s")
    wid = sid * NC + cid
    b = wid
    io = _io()
    z16 = jnp.zeros((L,), jnp.int32)
    preds = (lp, mp, sp)

    pltpu.sync_copy(bxh.at[pl.ds(b * 80, 80)], bx)
    pltpu.sync_copy(lbh.at[pl.ds(b * 32, 32)], lbv)

    def dense_phase(pref, Ns, chunk, nv, K):
        row0 = wid * chunk

        def fill(i, _):
            row = jnp.minimum(row0 + i * L + io, Ns - 1)
            idxd[pl.ds(i * L, L)] = row * 128 + 4
            return 0
        lax.fori_loop(0, nv, fill, 0)
        for j in range(nv * L, K * 128, L):
            idxd[pl.ds(j, L)] = z16 + 4
        cps = []
        for j in range(K):
            cp = pltpu.make_async_copy(
                pref.at[idxd.at[pl.ds(j * 128, 128)]],
                dg.at[pl.ds(j * 128, 128)], semd)
            cp.start()
            cps.append(cp)
        for cp in cps:
            cp.wait()

        def red(i, acc):
            v = dg[pl.ds(i * L, L)]
            msk = (i * L + io) < chunk
            return acc + jnp.where(msk, _softplus(v), 0.0)
        return lax.fori_loop(0, nv, red, jnp.zeros((L,), jnp.float32))

    def sparse_phase(pref, G, anc, accs):
        fG = jnp.float32(G)
        for c in range(2):
            j = jnp.minimum(c * L + io, NB - 1)
            valid = (c * L + io) < NB
            x = plsc.load_gather(bx, [j * 4])
            y = plsc.load_gather(bx, [j * 4 + 1])
            w = plsc.load_gather(bx, [j * 4 + 2])
            h = plsc.load_gather(bx, [j * 4 + 3])
            lab = lbv[pl.ds(c * L, L)]
            gx = jnp.minimum((x * fG).astype(jnp.int32), G - 1)
            gy = jnp.minimum((y * fG).astype(jnp.int32), G - 1)
            bw = w * fG
            bh = h * fG
            best = jnp.zeros((L,), jnp.float32) - 1.0
            a = z16
            for k in range(3):
                aw, ah = anc[k]
                inter = jnp.minimum(bw, aw) * jnp.minimum(bh, ah)
                iou = inter / (bw * bh + aw * ah - inter + 1e-16)
                upd = iou > best
                a = jnp.where(upd, k, a)
                best = jnp.where(upd, iou, best)
            aws = jnp.where(a == 0, anc[0][0], jnp.where(a == 1, anc[1][0], anc[2][0]))
            ahs = jnp.where(a == 0, anc[0][1], jnp.where(a == 1, anc[1][1], anc[2][1]))
            xc = x * fG - gx.astype(jnp.float32)
            yc = y * fG - gy.astype(jnp.float32)
            wc = _ln(bw / aws + 1e-16)
            hc = _ln(bh / ahs + 1e-16)
            cell = (a * G + gy) * G + gx
            cell_eff = jnp.where(valid, cell, -1 - (c * L + io))
            cellv[pl.ds(c * L, L)] = cell_eff
            lkeyv[pl.ds(c * L, L)] = cell_eff * 128 + lab
            basev[pl.ds(c * L, L)] = (cell * 32 + b) * 128
            tvf[pl.ds(c * L, L)] = xc
            tvf[pl.ds(48 + c * L, L)] = yc
            tvf[pl.ds(96 + c * L, L)] = wc
            tvf[pl.ds(144 + c * L, L)] = hc

        def dedup(n, _):
            c0 = cellv[pl.ds(0, L)]
            c1 = cellv[pl.ds(L, L)]
            k0 = lkeyv[pl.ds(0, L)]
            k1 = lkeyv[pl.ds(L, L)]
            cn = plsc.load_gather(cellv, [z16 + n])
            kn = plsc.load_gather(lkeyv, [z16 + n])
            lat0 = io > n
            lat1 = (io + L) > n
            coll = jnp.sum(jnp.where((c0 == cn) & lat0, 1, 0)
                           + jnp.where((c1 == cn) & lat1, 1, 0))
            lcoll = jnp.sum(jnp.where((k0 == kn) & lat0, 1, 0)
                            + jnp.where((k1 == kn) & lat1, 1, 0))
            m0 = io == 0
            plsc.store_scatter(winv, [z16 + n],
                               jnp.where(coll == 0, 1.0, 0.0) + jnp.zeros((L,), jnp.float32),
                               mask=m0)
            plsc.store_scatter(lwinv, [z16 + n],
                               jnp.where(lcoll == 0, 1.0, 0.0) + jnp.zeros((L,), jnp.float32),
                               mask=m0)
            return 0
        lax.fori_loop(0, NB, dedup, 0)

        def fill2(n, _):
            bn = plsc.load_gather(basev, [z16 + n])
            lb = plsc.load_gather(lbv, [z16 + n])
            for k in range(5):
                idxs[pl.ds(n * 96 + k * L, L)] = bn + io + k * L
            off5 = jnp.where(io < 5, io + 80, jnp.where(io == 5, 5 + lb, 84))
            idxs[pl.ds(n * 96 + 80, L)] = bn + off5
            return 0
        lax.fori_loop(0, NB, fill2, 0)

        cps = []
        for n in range(NB):
            cp = pltpu.make_async_copy(
                pref.at[idxs.at[pl.ds(n * 96, 96)]],
                gaths.at[pl.ds(n * 96, 96)], sems)
            cp.start()
            cps.append(cp)
        for cp in cps:
            cp.wait()

        def corr(n, accs):
            loc, obj, cls = accs
            g = [gaths[pl.ds(n * 96 + k * L, L)] for k in range(6)]
            s = [_softplus(gk) for gk in g]
            tv = plsc.load_gather(tvf, [jnp.minimum(io, 3) * 48 + n])
            wn = plsc.load_gather(winv, [z16 + n])
            lwn = plsc.load_gather(lwinv, [z16 + n])
            d = g[0] - tv
            loc = loc + wn * jnp.where(io < 4, d * d, 0.0)
            obj = obj + wn * jnp.where(io == 4, 0.5 * s[0] - g[0], 0.0)
            cb = (jnp.where(io >= 5, s[0], 0.0) + s[1] + s[2] + s[3] + s[4]
                  + jnp.where(io < 5, s[5], 0.0))
            cls = cls + wn * cb - lwn * jnp.where(io == 5, g[5], 0.0)
            return (loc, obj, cls)
        return lax.fori_loop(0, NB, corr, accs)

    zf = jnp.zeros((L,), jnp.float32)
    dense = zf
    accs = (zf, zf, zf)
    for si in range(3):
        G, Ns, ch, nv, K = SCALE[si]
        dense = dense + dense_phase(preds[si], Ns, ch, nv, K)
        accs = sparse_phase(preds[si], G, ANCH[si], accs)
    loc, obj, cls = accs
    obj = obj + 0.5 * dense
    ls = jnp.sum(loc)
    os_ = jnp.sum(obj)
    cs = jnp.sum(cls)
    outv[...] = (jnp.where(io == 0, ls, 0.0) + jnp.where(io == 1, os_, 0.0)
                 + jnp.where(io == 2, cs, 0.0))
    pltpu.sync_copy(outv, out.at[wid])


_MESH = plsc.VectorSubcoreMesh(core_axis_name="c", subcore_axis_name="s",
                               num_cores=NC, num_subcores=NS)

_kfn = pl.kernel(
    _body,
    out_type=jax.ShapeDtypeStruct((B, L), jnp.float32),
    mesh=_MESH,
    compiler_params=pltpu.CompilerParams(needs_layout_passes=False),
    scratch_types=[
        pltpu.VMEM((8192,), jnp.int32),
        pltpu.VMEM((8192,), jnp.float32),
        pltpu.VMEM((1920,), jnp.int32),
        pltpu.VMEM((1920,), jnp.float32),
        pltpu.VMEM((80,), jnp.float32),
        pltpu.VMEM((32,), jnp.int32),
        pltpu.VMEM((32,), jnp.int32),
        pltpu.VMEM((32,), jnp.int32),
        pltpu.VMEM((32,), jnp.int32),
        pltpu.VMEM((192,), jnp.float32),
        pltpu.VMEM((32,), jnp.float32),
        pltpu.VMEM((32,), jnp.float32),
        pltpu.VMEM((L,), jnp.float32),
        pltpu.SemaphoreType.DMA,
        pltpu.SemaphoreType.DMA,
    ],
)


def _phys(p):
    pt = jnp.transpose(p, (1, 2, 3, 0, 4))
    pp = jnp.pad(pt, ((0, 0), (0, 0), (0, 0), (0, 0), (0, 43)))
    return pp.reshape(-1)


def kernel(large_pred, medium_pred, small_pred, boxes, labels, scale_mask):
    lpf = _phys(large_pred)
    mpf = _phys(medium_pred)
    spf = _phys(small_pred)
    bxf = boxes.reshape(-1)
    lbp = jnp.zeros((B, 32), jnp.int32).at[:, :NB].set(
        labels.astype(jnp.int32)).reshape(-1)
    parts = _kfn(lpf, mpf, spf, bxf, lbp)
    loc = jnp.sum(parts[:, 0])
    obj = jnp.sum(parts[:, 1])
    cls = jnp.sum(parts[:, 2])
    total = 5.0 * loc + obj + cls
    return jnp.stack([total, loc, obj, cls])


# speedup vs baseline: 15.0212x; 1.0592x over previous
import jax
import jax.numpy as jnp
from jax import lax
from jax.experimental import pallas as pl
from jax.experimental.pallas import tpu as pltpu
from jax.experimental.pallas import tpu_sc as plsc

L = 16
NC, NS = 2, 16
B, NB, C = 32, 20, 80
GRIDS = (13, 26, 52)
ANCH = (
    ((3.625, 2.8125), (4.875, 6.1875), (11.65625, 10.1875)),
    ((1.875, 3.8125), (3.875, 2.8125), (3.6875, 7.4375)),
    ((1.25, 1.625), (2.0, 3.75), (4.125, 2.875)),
)
SCALE = []
for _G in GRIDS:
    _N = 3 * _G * _G * B
    _ch = _N // 32
    SCALE.append((_G, _N, _ch, (_ch + L - 1) // L, (_ch + 127) // 128))

LN2 = 0.6931471805599453
SQRTH = 1.41421356237
_LOGP = (7.0376836292e-2, -1.1514610310e-1, 1.1676998740e-1, -1.2420140846e-1,
         1.4249322787e-1, -1.6668057665e-1, 2.0000714765e-1, -2.4999993993e-1,
         3.3333331174e-1)


def _io():
    return lax.iota(jnp.int32, L)


def _ln(v):
    bits = lax.bitcast_convert_type(v, jnp.int32)
    e = lax.shift_right_arithmetic(bits, 23) - 127
    m = lax.bitcast_convert_type(
        (bits & jnp.int32(0x007FFFFF)) | jnp.int32(0x3F800000), jnp.float32)
    big = m > SQRTH
    m = jnp.where(big, m * 0.5, m)
    e = e + jnp.where(big, 1, 0)
    t = m - 1.0
    p = jnp.full((L,), _LOGP[0], jnp.float32)
    for c in _LOGP[1:]:
        p = p * t + c
    y = t + t * t * (t * p - 0.5)
    return e.astype(jnp.float32) * LN2 + y


def _softplus(x):
    return jnp.maximum(x, 0.0) + _ln(1.0 + jnp.exp(-jnp.abs(x)))


def _body(lp, mp, sp, bxh, lbh, out,
          idxdL, idxdM, idxdS, dgL, dgM, dgS,
          idxsL, idxsM, idxsS, gathsL, gathsM, gathsS,
          bx, lbv, cellv, lkeyv, basev, tvf,
          winv, lwinv, outv, semd, sems):
    cid = lax.axis_index("c")
    sid = lax.axis_index("s")
    wid = sid * NC + cid
    b = wid
    io = _io()
    z16 = jnp.zeros((L,), jnp.int32)
    preds = (lp, mp, sp)
    idxds = (idxdL, idxdM, idxdS)
    dgs = (dgL, dgM, dgS)
    idxss = (idxsL, idxsM, idxsS)
    gathss = (gathsL, gathsM, gathsS)

    pltpu.sync_copy(bxh.at[pl.ds(b * 80, 80)], bx)
    pltpu.sync_copy(lbh.at[pl.ds(b * 32, 32)], lbv)

    def dense_fire(pref, Ns, chunk, nv, K, idxd, dg):
        row0 = wid * chunk

        def fill(i, _):
            row = jnp.minimum(row0 + i * L + io, Ns - 1)
            idxd[pl.ds(i * L, L)] = row * 128 + 4
            return 0
        lax.fori_loop(0, nv, fill, 0)
        for j in range(nv * L, K * 128, L):
            idxd[pl.ds(j, L)] = z16 + 4
        cps = []
        for j in range(K):
            cp = pltpu.make_async_copy(
                pref.at[idxd.at[pl.ds(j * 128, 128)]],
                dg.at[pl.ds(j * 128, 128)], semd)
            cp.start()
            cps.append(cp)
        return cps

    def dense_reduce(chunk, nv, dg, cps, acc):
        for cp in cps:
            cp.wait()

        def red(i, acc):
            v = dg[pl.ds(i * L, L)]
            msk = (i * L + io) < chunk
            return acc + jnp.where(msk, _softplus(v), 0.0)
        return lax.fori_loop(0, nv, red, acc)

    def sparse_prep(pref, G, anc, si, idxs, gaths):
        fG = jnp.float32(G)
        for c in range(2):
            j = jnp.minimum(c * L + io, NB - 1)
            valid = (c * L + io) < NB
            x = plsc.load_gather(bx, [j * 4])
            y = plsc.load_gather(bx, [j * 4 + 1])
            w = plsc.load_gather(bx, [j * 4 + 2])
            h = plsc.load_gather(bx, [j * 4 + 3])
            lab = lbv[pl.ds(c * L, L)]
            gx = jnp.minimum((x * fG).astype(jnp.int32), G - 1)
            gy = jnp.minimum((y * fG).astype(jnp.int32), G - 1)
            bw = w * fG
            bh = h * fG
            best = jnp.zeros((L,), jnp.float32) - 1.0
            a = z16
            for k in range(3):
                aw, ah = anc[k]
                inter = jnp.minimum(bw, aw) * jnp.minimum(bh, ah)
                iou = inter / (bw * bh + aw * ah - inter + 1e-16)
                upd = iou > best
                a = jnp.where(upd, k, a)
                best = jnp.where(upd, iou, best)
            aws = jnp.where(a == 0, anc[0][0], jnp.where(a == 1, anc[1][0], anc[2][0]))
            ahs = jnp.where(a == 0, anc[0][1], jnp.where(a == 1, anc[1][1], anc[2][1]))
            xc = x * fG - gx.astype(jnp.float32)
            yc = y * fG - gy.astype(jnp.float32)
            wc = _ln(bw / aws + 1e-16)
            hc = _ln(bh / ahs + 1e-16)
            cell = (a * G + gy) * G + gx
            cell_eff = jnp.where(valid, cell, -1 - (c * L + io))
            cellv[pl.ds(c * L, L)] = cell_eff
            lkeyv[pl.ds(c * L, L)] = cell_eff * 128 + lab
            basev[pl.ds(c * L, L)] = (cell * 32 + b) * 128
            tvf[pl.ds(si * 192 + c * L, L)] = xc
            tvf[pl.ds(si * 192 + 48 + c * L, L)] = yc
            tvf[pl.ds(si * 192 + 96 + c * L, L)] = wc
            tvf[pl.ds(si * 192 + 144 + c * L, L)] = hc

        def fill2(n, _):
            bn = plsc.load_gather(basev, [z16 + n])
            lb = plsc.load_gather(lbv, [z16 + n])
            for k in range(5):
                idxs[pl.ds(n * 96 + k * L, L)] = bn + io + k * L
            off5 = jnp.where(io < 5, io + 80, jnp.where(io == 5, 5 + lb, 84))
            idxs[pl.ds(n * 96 + 80, L)] = bn + off5
            return 0
        lax.fori_loop(0, NB, fill2, 0)

        cps = []
        for n in range(NB):
            cp = pltpu.make_async_copy(
                pref.at[idxs.at[pl.ds(n * 96, 96)]],
                gaths.at[pl.ds(n * 96, 96)], sems)
            cp.start()
            cps.append(cp)

        def dedup(n, _):
            c0 = cellv[pl.ds(0, L)]
            c1 = cellv[pl.ds(L, L)]
            k0 = lkeyv[pl.ds(0, L)]
            k1 = lkeyv[pl.ds(L, L)]
            cn = plsc.load_gather(cellv, [z16 + n])
            kn = plsc.load_gather(lkeyv, [z16 + n])
            lat0 = io > n
            lat1 = (io + L) > n
            coll = jnp.sum(jnp.where((c0 == cn) & lat0, 1, 0)
                           + jnp.where((c1 == cn) & lat1, 1, 0))
            lcoll = jnp.sum(jnp.where((k0 == kn) & lat0, 1, 0)
                            + jnp.where((k1 == kn) & lat1, 1, 0))
            m0 = io == 0
            plsc.store_scatter(winv, [z16 + si * 32 + n],
                               jnp.where(coll == 0, 1.0, 0.0) + jnp.zeros((L,), jnp.float32),
                               mask=m0)
            plsc.store_scatter(lwinv, [z16 + si * 32 + n],
                               jnp.where(lcoll == 0, 1.0, 0.0) + jnp.zeros((L,), jnp.float32),
                               mask=m0)
            return 0
        lax.fori_loop(0, NB, dedup, 0)
        return cps

    def sparse_corr(si, gaths, cps, accs):
        for cp in cps:
            cp.wait()

        def corr(n, accs):
            loc, obj, cls = accs
            g = [gaths[pl.ds(n * 96 + k * L, L)] for k in range(6)]
            s = [_softplus(gk) for gk in g]
            tv = plsc.load_gather(tvf, [si * 192 + jnp.minimum(io, 3) * 48 + n])
            wn = plsc.load_gather(winv, [z16 + si * 32 + n])
            lwn = plsc.load_gather(lwinv, [z16 + si * 32 + n])
            d = g[0] - tv
            loc = loc + wn * jnp.where(io < 4, d * d, 0.0)
            obj = obj + wn * jnp.where(io == 4, 0.5 * s[0] - g[0], 0.0)
            cb = (jnp.where(io >= 5, s[0], 0.0) + s[1] + s[2] + s[3] + s[4]
                  + jnp.where(io < 5, s[5], 0.0))
            cls = cls + wn * cb - lwn * jnp.where(io == 5, g[5], 0.0)
            return (loc, obj, cls)
        return lax.fori_loop(0, NB, corr, accs)

    dense_cps = []
    for si in range(3):
        G, Ns, ch, nv, K = SCALE[si]
        dense_cps.append(dense_fire(preds[si], Ns, ch, nv, K, idxds[si], dgs[si]))
    sparse_cps = []
    for si in range(3):
        G, Ns, ch, nv, K = SCALE[si]
        sparse_cps.append(sparse_prep(preds[si], G, ANCH[si], si, idxss[si], gathss[si]))

    zf = jnp.zeros((L,), jnp.float32)
    accs = (zf, zf, zf)
    for si in range(3):
        accs = sparse_corr(si, gathss[si], sparse_cps[si], accs)
    dense = zf
    for si in range(3):
        G, Ns, ch, nv, K = SCALE[si]
        dense = dense_reduce(ch, nv, dgs[si], dense_cps[si], dense)

    loc, obj, cls = accs
    obj = obj + 0.5 * dense
    ls = jnp.sum(loc)
    os_ = jnp.sum(obj)
    cs = jnp.sum(cls)
    outv[...] = (jnp.where(io == 0, ls, 0.0) + jnp.where(io == 1, os_, 0.0)
                 + jnp.where(io == 2, cs, 0.0))
    pltpu.sync_copy(outv, out.at[wid])


_MESH = plsc.VectorSubcoreMesh(core_axis_name="c", subcore_axis_name="s",
                               num_cores=NC, num_subcores=NS)

_kfn = pl.kernel(
    _body,
    out_type=jax.ShapeDtypeStruct((B, L), jnp.float32),
    mesh=_MESH,
    compiler_params=pltpu.CompilerParams(needs_layout_passes=False),
    scratch_types=[
        pltpu.VMEM((512,), jnp.int32),
        pltpu.VMEM((2048,), jnp.int32),
        pltpu.VMEM((8192,), jnp.int32),
        pltpu.VMEM((512,), jnp.float32),
        pltpu.VMEM((2048,), jnp.float32),
        pltpu.VMEM((8192,), jnp.float32),
        pltpu.VMEM((1920,), jnp.int32),
        pltpu.VMEM((1920,), jnp.int32),
        pltpu.VMEM((1920,), jnp.int32),
        pltpu.VMEM((1920,), jnp.float32),
        pltpu.VMEM((1920,), jnp.float32),
        pltpu.VMEM((1920,), jnp.float32),
        pltpu.VMEM((80,), jnp.float32),
        pltpu.VMEM((32,), jnp.int32),
        pltpu.VMEM((32,), jnp.int32),
        pltpu.VMEM((32,), jnp.int32),
        pltpu.VMEM((32,), jnp.int32),
        pltpu.VMEM((576,), jnp.float32),
        pltpu.VMEM((96,), jnp.float32),
        pltpu.VMEM((96,), jnp.float32),
        pltpu.VMEM((L,), jnp.float32),
        pltpu.SemaphoreType.DMA,
        pltpu.SemaphoreType.DMA,
    ],
)


def _phys(p):
    pt = jnp.transpose(p, (1, 2, 3, 0, 4))
    pp = jnp.pad(pt, ((0, 0), (0, 0), (0, 0), (0, 0), (0, 43)))
    return pp.reshape(-1)


def kernel(large_pred, medium_pred, small_pred, boxes, labels, scale_mask):
    lpf = _phys(large_pred)
    mpf = _phys(medium_pred)
    spf = _phys(small_pred)
    bxf = boxes.reshape(-1)
    lbp = jnp.zeros((B, 32), jnp.int32).at[:, :NB].set(
        labels.astype(jnp.int32)).reshape(-1)
    parts = _kfn(lpf, mpf, spf, bxf, lbp)
    loc = jnp.sum(parts[:, 0])
    obj = jnp.sum(parts[:, 1])
    cls = jnp.sum(parts[:, 2])
    total = 5.0 * loc + obj + cls
    return jnp.stack([total, loc, obj, cls])


# speedup vs baseline: 15.1653x; 1.0096x over previous
import jax
import jax.numpy as jnp
from jax import lax
from jax.experimental import pallas as pl
from jax.experimental.pallas import tpu as pltpu
from jax.experimental.pallas import tpu_sc as plsc

L = 16
NC, NS = 2, 16
B, NB, C = 32, 20, 80
GRIDS = (13, 26, 52)
ANCH = (
    ((3.625, 2.8125), (4.875, 6.1875), (11.65625, 10.1875)),
    ((1.875, 3.8125), (3.875, 2.8125), (3.6875, 7.4375)),
    ((1.25, 1.625), (2.0, 3.75), (4.125, 2.875)),
)
SCALE = []
for _G in GRIDS:
    _N = 3 * _G * _G * B
    _ch = _N // 32
    SCALE.append((_G, _N, _ch, (_ch + L - 1) // L, (_ch + 127) // 128))

LN2 = 0.6931471805599453
SQRTH = 1.41421356237
_LOGP = (7.0376836292e-2, -1.1514610310e-1, 1.1676998740e-1, -1.2420140846e-1,
         1.4249322787e-1, -1.6668057665e-1, 2.0000714765e-1, -2.4999993993e-1,
         3.3333331174e-1)


def _io():
    return lax.iota(jnp.int32, L)


def _ln(v):
    bits = lax.bitcast_convert_type(v, jnp.int32)
    e = lax.shift_right_arithmetic(bits, 23) - 127
    m = lax.bitcast_convert_type(
        (bits & jnp.int32(0x007FFFFF)) | jnp.int32(0x3F800000), jnp.float32)
    big = m > SQRTH
    m = jnp.where(big, m * 0.5, m)
    e = e + jnp.where(big, 1, 0)
    t = m - 1.0
    p = jnp.full((L,), _LOGP[0], jnp.float32)
    for c in _LOGP[1:]:
        p = p * t + c
    y = t + t * t * (t * p - 0.5)
    return e.astype(jnp.float32) * LN2 + y


def _softplus(x):
    return jnp.maximum(x, 0.0) + _ln(1.0 + jnp.exp(-jnp.abs(x)))


def _make_body(sis):
    nsc = len(sis)

    def _body(*refs):
        preds = refs[:nsc]
        bxh, lbh, out = refs[nsc:nsc + 3]
        r = nsc + 3
        idxds = refs[r:r + nsc]; r += nsc
        dgs = refs[r:r + nsc]; r += nsc
        idxss = refs[r:r + nsc]; r += nsc
        gathss = refs[r:r + nsc]; r += nsc
        (bx, lbv, cellv, lkeyv, basev, tvf,
         winv, lwinv, outv, semd, sems) = refs[r:]
        cid = lax.axis_index("c")
        sid = lax.axis_index("s")
        wid = sid * NC + cid
        b = wid
        io = _io()
        z16 = jnp.zeros((L,), jnp.int32)

        pltpu.sync_copy(bxh.at[pl.ds(b * 80, 80)], bx)
        pltpu.sync_copy(lbh.at[pl.ds(b * 32, 32)], lbv)

        def dense_fire(pref, Ns, chunk, nv, K, idxd, dg):
            row0 = wid * chunk

            def fill(i, _):
                row = jnp.minimum(row0 + i * L + io, Ns - 1)
                idxd[pl.ds(i * L, L)] = row * 128 + 4
                return 0
            lax.fori_loop(0, nv, fill, 0)
            for j in range(nv * L, K * 128, L):
                idxd[pl.ds(j, L)] = z16 + 4
            cps = []
            for j in range(K):
                cp = pltpu.make_async_copy(
                    pref.at[idxd.at[pl.ds(j * 128, 128)]],
                    dg.at[pl.ds(j * 128, 128)], semd)
                cp.start()
                cps.append(cp)
            return cps

        def dense_reduce(chunk, nv, dg, cps, acc):
            for cp in cps:
                cp.wait()

            def red(i, acc):
                v = dg[pl.ds(i * L, L)]
                msk = (i * L + io) < chunk
                return acc + jnp.where(msk, _softplus(v), 0.0)
            return lax.fori_loop(0, nv, red, acc)

        def sparse_prep(pref, G, anc, sk, idxs, gaths):
            fG = jnp.float32(G)
            for c in range(2):
                j = jnp.minimum(c * L + io, NB - 1)
                valid = (c * L + io) < NB
                x = plsc.load_gather(bx, [j * 4])
                y = plsc.load_gather(bx, [j * 4 + 1])
                w = plsc.load_gather(bx, [j * 4 + 2])
                h = plsc.load_gather(bx, [j * 4 + 3])
                lab = lbv[pl.ds(c * L, L)]
                gx = jnp.minimum((x * fG).astype(jnp.int32), G - 1)
                gy = jnp.minimum((y * fG).astype(jnp.int32), G - 1)
                bw = w * fG
                bh = h * fG
                best = jnp.zeros((L,), jnp.float32) - 1.0
                a = z16
                for k in range(3):
                    aw, ah = anc[k]
                    inter = jnp.minimum(bw, aw) * jnp.minimum(bh, ah)
                    iou = inter / (bw * bh + aw * ah - inter + 1e-16)
                    upd = iou > best
                    a = jnp.where(upd, k, a)
                    best = jnp.where(upd, iou, best)
                aws = jnp.where(a == 0, anc[0][0],
                                jnp.where(a == 1, anc[1][0], anc[2][0]))
                ahs = jnp.where(a == 0, anc[0][1],
                                jnp.where(a == 1, anc[1][1], anc[2][1]))
                xc = x * fG - gx.astype(jnp.float32)
                yc = y * fG - gy.astype(jnp.float32)
                wc = _ln(bw / aws + 1e-16)
                hc = _ln(bh / ahs + 1e-16)
                cell = (a * G + gy) * G + gx
                cell_eff = jnp.where(valid, cell, -1 - (c * L + io))
                cellv[pl.ds(c * L, L)] = cell_eff
                lkeyv[pl.ds(c * L, L)] = cell_eff * 128 + lab
                basev[pl.ds(c * L, L)] = (cell * 32 + b) * 128
                tvf[pl.ds(sk * 192 + c * L, L)] = xc
                tvf[pl.ds(sk * 192 + 48 + c * L, L)] = yc
                tvf[pl.ds(sk * 192 + 96 + c * L, L)] = wc
                tvf[pl.ds(sk * 192 + 144 + c * L, L)] = hc

            def fill2(n, _):
                bn = plsc.load_gather(basev, [z16 + n])
                lb = plsc.load_gather(lbv, [z16 + n])
                for k in range(5):
                    idxs[pl.ds(n * 96 + k * L, L)] = bn + io + k * L
                off5 = jnp.where(io < 5, io + 80,
                                 jnp.where(io == 5, 5 + lb, 84))
                idxs[pl.ds(n * 96 + 80, L)] = bn + off5
                return 0
            lax.fori_loop(0, NB, fill2, 0)

            cps = []
            for n in range(NB):
                cp = pltpu.make_async_copy(
                    pref.at[idxs.at[pl.ds(n * 96, 96)]],
                    gaths.at[pl.ds(n * 96, 96)], sems)
                cp.start()
                cps.append(cp)

            def dedup(n, _):
                c0 = cellv[pl.ds(0, L)]
                c1 = cellv[pl.ds(L, L)]
                k0 = lkeyv[pl.ds(0, L)]
                k1 = lkeyv[pl.ds(L, L)]
                cn = plsc.load_gather(cellv, [z16 + n])
                kn = plsc.load_gather(lkeyv, [z16 + n])
                lat0 = io > n
                lat1 = (io + L) > n
                coll = jnp.sum(jnp.where((c0 == cn) & lat0, 1, 0)
                               + jnp.where((c1 == cn) & lat1, 1, 0))
                lcoll = jnp.sum(jnp.where((k0 == kn) & lat0, 1, 0)
                                + jnp.where((k1 == kn) & lat1, 1, 0))
                m0 = io == 0
                plsc.store_scatter(
                    winv, [z16 + sk * 32 + n],
                    jnp.where(coll == 0, 1.0, 0.0) + jnp.zeros((L,), jnp.float32),
                    mask=m0)
                plsc.store_scatter(
                    lwinv, [z16 + sk * 32 + n],
                    jnp.where(lcoll == 0, 1.0, 0.0) + jnp.zeros((L,), jnp.float32),
                    mask=m0)
                return 0
            lax.fori_loop(0, NB, dedup, 0)
            return cps

        def sparse_corr(sk, gaths, cps, accs):
            for cp in cps:
                cp.wait()

            def corr(n, accs):
                loc, obj, cls = accs
                g = [gaths[pl.ds(n * 96 + k * L, L)] for k in range(6)]
                s = [_softplus(gk) for gk in g]
                tv = plsc.load_gather(
                    tvf, [sk * 192 + jnp.minimum(io, 3) * 48 + n])
                wn = plsc.load_gather(winv, [z16 + sk * 32 + n])
                lwn = plsc.load_gather(lwinv, [z16 + sk * 32 + n])
                d = g[0] - tv
                loc = loc + wn * jnp.where(io < 4, d * d, 0.0)
                obj = obj + wn * jnp.where(io == 4, 0.5 * s[0] - g[0], 0.0)
                cb = (jnp.where(io >= 5, s[0], 0.0) + s[1] + s[2] + s[3] + s[4]
                      + jnp.where(io < 5, s[5], 0.0))
                cls = cls + wn * cb - lwn * jnp.where(io == 5, g[5], 0.0)
                return (loc, obj, cls)
            return lax.fori_loop(0, NB, corr, accs)

        dense_cps = []
        for k, si in enumerate(sis):
            G, Ns, ch, nv, K = SCALE[si]
            dense_cps.append(
                dense_fire(preds[k], Ns, ch, nv, K, idxds[k], dgs[k]))
        sparse_cps = []
        for k, si in enumerate(sis):
            G, Ns, ch, nv, K = SCALE[si]
            sparse_cps.append(
                sparse_prep(preds[k], G, ANCH[si], k, idxss[k], gathss[k]))

        zf = jnp.zeros((L,), jnp.float32)
        accs = (zf, zf, zf)
        for k, si in enumerate(sis):
            accs = sparse_corr(k, gathss[k], sparse_cps[k], accs)
        dense = zf
        for k, si in enumerate(sis):
            G, Ns, ch, nv, K = SCALE[si]
            dense = dense_reduce(ch, nv, dgs[k], dense_cps[k], dense)

        loc, obj, cls = accs
        obj = obj + 0.5 * dense
        ls = jnp.sum(loc)
        os_ = jnp.sum(obj)
        cs = jnp.sum(cls)
        outv[...] = (jnp.where(io == 0, ls, 0.0) + jnp.where(io == 1, os_, 0.0)
                     + jnp.where(io == 2, cs, 0.0))
        pltpu.sync_copy(outv, out.at[wid])

    return _body


_MESH = plsc.VectorSubcoreMesh(core_axis_name="c", subcore_axis_name="s",
                               num_cores=NC, num_subcores=NS)


def _make_kernel(sis):
    nsc = len(sis)
    scratches = []
    for si in sis:
        K = SCALE[si][4]
        scratches.append(pltpu.VMEM((K * 128,), jnp.int32))
    for si in sis:
        K = SCALE[si][4]
        scratches.append(pltpu.VMEM((K * 128,), jnp.float32))
    scratches += [pltpu.VMEM((1920,), jnp.int32) for _ in sis]
    scratches += [pltpu.VMEM((1920,), jnp.float32) for _ in sis]
    scratches += [
        pltpu.VMEM((80,), jnp.float32),
        pltpu.VMEM((32,), jnp.int32),
        pltpu.VMEM((32,), jnp.int32),
        pltpu.VMEM((32,), jnp.int32),
        pltpu.VMEM((32,), jnp.int32),
        pltpu.VMEM((192 * nsc,), jnp.float32),
        pltpu.VMEM((32 * nsc,), jnp.float32),
        pltpu.VMEM((32 * nsc,), jnp.float32),
        pltpu.VMEM((L,), jnp.float32),
        pltpu.SemaphoreType.DMA,
        pltpu.SemaphoreType.DMA,
    ]
    return pl.kernel(
        _make_body(sis),
        out_type=jax.ShapeDtypeStruct((B, L), jnp.float32),
        mesh=_MESH,
        compiler_params=pltpu.CompilerParams(needs_layout_passes=False),
        scratch_types=scratches,
    )


_kfnLM = _make_kernel((0, 1))
_kfnS = _make_kernel((2,))


def _phys(p):
    pt = jnp.transpose(p, (1, 2, 3, 0, 4))
    pp = jnp.pad(pt, ((0, 0), (0, 0), (0, 0), (0, 0), (0, 43)))
    return pp.reshape(-1)


def kernel(large_pred, medium_pred, small_pred, boxes, labels, scale_mask):
    lpf = _phys(large_pred)
    mpf = _phys(medium_pred)
    spf = _phys(small_pred)
    bxf = boxes.reshape(-1)
    lbp = jnp.zeros((B, 32), jnp.int32).at[:, :NB].set(
        labels.astype(jnp.int32)).reshape(-1)
    partsA = _kfnLM(lpf, mpf, bxf, lbp)
    partsB = _kfnS(spf, bxf, lbp)
    parts = partsA + partsB
    loc = jnp.sum(parts[:, 0])
    obj = jnp.sum(parts[:, 1])
    cls = jnp.sum(parts[:, 2])
    total = 5.0 * loc + obj + cls
    return jnp.stack([total, loc, obj, cls])
